# baseline (speedup 1.0000x reference)
import functools

import jax
import jax.numpy as jnp
from jax import lax
from jax.experimental import pallas as pl
from jax.experimental.pallas import tpu as pltpu
from jax.experimental.pallas import tpu_sc as plsc

VOCAB = 1000
DIM = 2
WINDOW = 5
BATCH = 16384

_LANES = 16


def _make_sc_gather():
  info = plsc.get_sparse_core_info()
  nc, ns = info.num_cores, info.num_subcores
  nw = nc * ns
  b_per_w = BATCH // nw
  mesh = plsc.VectorSubcoreMesh(core_axis_name="c", subcore_axis_name="s")

  @functools.partial(
      pl.kernel,
      mesh=mesh,
      compiler_params=pltpu.CompilerParams(needs_layout_passes=False),
      out_type=jax.ShapeDtypeStruct((DIM * BATCH,), jnp.float32),
      scratch_types=[
          pltpu.VMEM((b_per_w,), jnp.int32),
          pltpu.VMEM((VOCAB * DIM,), jnp.float32),
          pltpu.VMEM((b_per_w,), jnp.float32),
          pltpu.VMEM((b_per_w,), jnp.float32),
          pltpu.SemaphoreType.DMA,
          pltpu.SemaphoreType.DMA,
      ],
  )
  def sc_gather(x_hbm, tab_hbm, e_hbm, x_v, tab_v, e0_v, e1_v, sem0, sem1):
    wid = lax.axis_index("s") * nc + lax.axis_index("c")
    base = wid * b_per_w
    cp_x = pltpu.async_copy(x_hbm.at[pl.ds(base, b_per_w)], x_v, sem0)
    cp_t = pltpu.async_copy(tab_hbm, tab_v, sem1)
    cp_x.wait()
    cp_t.wait()
    for i in range(b_per_w // _LANES):
      idx = x_v[pl.ds(i * _LANES, _LANES)]
      e0_v[pl.ds(i * _LANES, _LANES)] = plsc.load_gather(tab_v, [idx * DIM])
      e1_v[pl.ds(i * _LANES, _LANES)] = plsc.load_gather(tab_v, [idx * DIM + 1])
    cp_e0 = pltpu.async_copy(e0_v, e_hbm.at[pl.ds(base, b_per_w)], sem0)
    cp_e1 = pltpu.async_copy(e1_v, e_hbm.at[pl.ds(BATCH + base, b_per_w)], sem1)
    cp_e0.wait()
    cp_e1.wait()

  return sc_gather


_sc_gather = _make_sc_gather()

_BV = 48


def _tc_project(e_ref, w_ref, out_ref):
  e0 = e_ref[pl.ds(0, BATCH)].reshape(1, BATCH)
  e1 = e_ref[pl.ds(BATCH, BATCH)].reshape(1, BATCH)
  logits = (w_ref[:, 0:1] * e0 + w_ref[:, 1:2] * e1
            + w_ref[:, 2:3])
  out_ref[...] = jnp.broadcast_to(logits[None], (WINDOW, _BV, BATCH))


def kernel(x, emb_table, W, b):
  e = _sc_gather(x.astype(jnp.int32), emb_table.reshape(-1))
  waug = jnp.concatenate([W, b.reshape(VOCAB, 1)], axis=1)
  out_t = pl.pallas_call(
      _tc_project,
      grid=(pl.cdiv(VOCAB, _BV),),
      in_specs=[
          pl.BlockSpec((DIM * BATCH,), lambda i: (0,)),
          pl.BlockSpec((_BV, DIM + 1), lambda i: (i, 0)),
      ],
      out_specs=pl.BlockSpec((WINDOW, _BV, BATCH), lambda i: (0, i, 0)),
      out_shape=jax.ShapeDtypeStruct((WINDOW, VOCAB, BATCH), jnp.float32),
  )(e, waug)
  return jnp.transpose(out_t, (2, 0, 1))


# speedup vs baseline: 1.0103x; 1.0103x over previous
import functools

import jax
import jax.numpy as jnp
from jax import lax
from jax.experimental import pallas as pl
from jax.experimental.pallas import tpu as pltpu
from jax.experimental.pallas import tpu_sc as plsc

VOCAB = 1000
DIM = 2
WINDOW = 5
BATCH = 16384

_LANES = 16


def _make_sc_gather():
  info = plsc.get_sparse_core_info()
  nc, ns = info.num_cores, info.num_subcores
  nw = nc * ns
  b_per_w = BATCH // nw
  mesh = plsc.VectorSubcoreMesh(core_axis_name="c", subcore_axis_name="s")

  @functools.partial(
      pl.kernel,
      mesh=mesh,
      compiler_params=pltpu.CompilerParams(needs_layout_passes=False),
      out_type=jax.ShapeDtypeStruct((DIM * BATCH,), jnp.float32),
      scratch_types=[
          pltpu.VMEM((b_per_w,), jnp.int32),
          pltpu.VMEM((VOCAB * DIM,), jnp.float32),
          pltpu.VMEM((b_per_w,), jnp.float32),
          pltpu.VMEM((b_per_w,), jnp.float32),
          pltpu.SemaphoreType.DMA,
          pltpu.SemaphoreType.DMA,
      ],
  )
  def sc_gather(x_hbm, tab_hbm, e_hbm, x_v, tab_v, e0_v, e1_v, sem0, sem1):
    wid = lax.axis_index("s") * nc + lax.axis_index("c")
    base = wid * b_per_w
    cp_x = pltpu.async_copy(x_hbm.at[pl.ds(base, b_per_w)], x_v, sem0)
    cp_t = pltpu.async_copy(tab_hbm, tab_v, sem1)
    cp_x.wait()
    cp_t.wait()
    def body(i, _):
      off = i * _LANES
      idx = x_v[pl.ds(off, _LANES)]
      e0_v[pl.ds(off, _LANES)] = plsc.load_gather(tab_v, [idx * DIM])
      e1_v[pl.ds(off, _LANES)] = plsc.load_gather(tab_v, [idx * DIM + 1])
      return _

    lax.fori_loop(0, b_per_w // _LANES, body, None)
    cp_e0 = pltpu.async_copy(e0_v, e_hbm.at[pl.ds(base, b_per_w)], sem0)
    cp_e1 = pltpu.async_copy(e1_v, e_hbm.at[pl.ds(BATCH + base, b_per_w)], sem1)
    cp_e0.wait()
    cp_e1.wait()

  return sc_gather


_sc_gather = _make_sc_gather()

_BV = 40


def _tc_project(e_ref, w_ref, out_ref):
  e0 = e_ref[pl.ds(0, BATCH)].reshape(1, BATCH)
  e1 = e_ref[pl.ds(BATCH, BATCH)].reshape(1, BATCH)
  logits = (w_ref[:, 0:1] * e0 + w_ref[:, 1:2] * e1
            + w_ref[:, 2:3])
  out_ref[...] = jnp.broadcast_to(logits[None], (WINDOW, _BV, BATCH))


def kernel(x, emb_table, W, b):
  e = _sc_gather(x.astype(jnp.int32), emb_table.reshape(-1))
  waug = jnp.concatenate([W, b.reshape(VOCAB, 1)], axis=1)
  out_t = pl.pallas_call(
      _tc_project,
      grid=(pl.cdiv(VOCAB, _BV),),
      in_specs=[
          pl.BlockSpec((DIM * BATCH,), lambda i: (0,)),
          pl.BlockSpec((_BV, DIM + 1), lambda i: (i, 0)),
      ],
      out_specs=pl.BlockSpec((WINDOW, _BV, BATCH), lambda i: (0, i, 0)),
      out_shape=jax.ShapeDtypeStruct((WINDOW, VOCAB, BATCH), jnp.float32),
  )(e, waug)
  return jnp.transpose(out_t, (2, 0, 1))
